# baseline (speedup 1.0000x reference)
import functools

import jax
import jax.numpy as jnp
import numpy as np
from jax.experimental import pallas as pl
from jax.experimental.pallas import tpu as pltpu

TOPK_N = 1024
_BN_SCALE = 1.0 / np.sqrt(1.0 + 1e-5)


def _conv(x, w, b=None, stride=1, pad=0, groups=1):
    out = jax.lax.conv_general_dilated(
        x, w, (stride, stride), [(pad, pad), (pad, pad)],
        dimension_numbers=('NCHW', 'OIHW', 'NCHW'), feature_group_count=groups)
    if b is not None:
        out = out + b[None, :, None, None]
    return out


def _basic(x, w, stride=1, pad=1):
    return jax.nn.relu(_conv(x, w, stride=stride, pad=pad) * _BN_SCALE)


def _unfold2d(x, ws):
    B, C, H, W = x.shape
    x = x.reshape(B, C, H // ws, ws, W // ws, ws)
    x = x.transpose(0, 1, 3, 5, 2, 4)
    return x.reshape(B, C * ws * ws, H // ws, W // ws)


def _resize_bilinear(x, H, W):
    return jax.image.resize(x, (x.shape[0], x.shape[1], H, W), method='bilinear')


def _backbone(x, p):
    xm = jnp.mean(x, axis=1, keepdims=True)
    mu = xm.mean(axis=(2, 3), keepdims=True)
    var = ((xm - mu) ** 2).mean(axis=(2, 3), keepdims=True)
    xn = (xm - mu) / jnp.sqrt(var + 1e-5)
    x1 = xn
    for w, s in zip(p['block1'], (1, 2, 1, 2)):
        x1 = _basic(x1, w, stride=s, pad=1)
    B, _, H, W = xn.shape
    sk = xn.reshape(B, 1, H // 4, 4, W // 4, 4).mean(axis=(3, 5))
    sk = _conv(sk, p['skip1_w'], p['skip1_b'])
    x2 = x1 + sk
    for w in p['block2']:
        x2 = _basic(x2, w, 1, 1)
    x3 = _basic(x2, p['block3'][0], 2, 1)
    x3 = _basic(x3, p['block3'][1], 1, 1)
    x3 = _basic(x3, p['block3'][2], 1, 0)
    x4 = _basic(x3, p['block4'][0], 2, 1)
    x4 = _basic(x4, p['block4'][1], 1, 1)
    x4 = _basic(x4, p['block4'][2], 1, 1)
    x5 = _basic(x4, p['block5'][0], 2, 1)
    x5 = _basic(x5, p['block5'][1], 1, 1)
    x5 = _basic(x5, p['block5'][2], 1, 1)
    x5 = _basic(x5, p['block5'][3], 1, 0)
    x4r = _resize_bilinear(x4, x3.shape[2], x3.shape[3])
    x5r = _resize_bilinear(x5, x3.shape[2], x3.shape[3])
    f = x3 + x4r + x5r
    f = _basic(f, p['fusion'][0], 1, 1)
    f = _basic(f, p['fusion'][1], 1, 1)
    feats = _conv(f, p['fusion_w'], p['fusion_b'])
    hh = _basic(feats, p['hm'][0], 1, 0)
    hh = _basic(hh, p['hm'][1], 1, 0)
    heatmap = jax.nn.sigmoid(_conv(hh, p['hm_w'], p['hm_b']))
    kk = _unfold2d(xn, 8)
    for w in p['kp']:
        kk = _basic(kk, w, 1, 0)
    kpts_logits = _conv(kk, p['kp_w'], p['kp_b'])
    return feats, kpts_logits, heatmap


def _gather_pts_xla(img, xi, yi):
    B, C, Hf, Wf = img.shape
    valid = (xi >= 0) & (xi < Wf) & (yi >= 0) & (yi < Hf)
    xc = jnp.clip(xi, 0, Wf - 1).astype(jnp.int32)
    yc = jnp.clip(yi, 0, Hf - 1).astype(jnp.int32)
    vals = jax.vmap(lambda im, y, x: im[:, y, x].T)(img, yc, xc)
    return vals * valid[..., None].astype(img.dtype)


def _grid_sample_pts_xla(img, pos, H, W, mode):
    B, C, Hf, Wf = img.shape
    gx = 2.0 * pos[..., 0] / (W - 1) - 1.0
    gy = 2.0 * pos[..., 1] / (H - 1) - 1.0
    ix = ((gx + 1.0) * Wf - 1.0) / 2.0
    iy = ((gy + 1.0) * Hf - 1.0) / 2.0
    if mode == 'nearest':
        return _gather_pts_xla(img, jnp.round(ix).astype(jnp.int32),
                               jnp.round(iy).astype(jnp.int32))
    x0 = jnp.floor(ix)
    y0 = jnp.floor(iy)
    tx = ix - x0
    ty = iy - y0
    x0i = x0.astype(jnp.int32)
    y0i = y0.astype(jnp.int32)
    v00 = _gather_pts_xla(img, x0i, y0i)
    v01 = _gather_pts_xla(img, x0i + 1, y0i)
    v10 = _gather_pts_xla(img, x0i, y0i + 1)
    v11 = _gather_pts_xla(img, x0i + 1, y0i + 1)
    return (v00 * ((1 - tx) * (1 - ty))[..., None] + v01 * (tx * (1 - ty))[..., None]
            + v10 * ((1 - tx) * ty)[..., None] + v11 * (tx * ty)[..., None])


def _kpts_heatmap(kpts_logits):
    scores = jax.nn.softmax(kpts_logits, axis=1)[:, :64]
    B, _, H, W = scores.shape
    hm = scores.transpose(0, 2, 3, 1).reshape(B, H, W, 8, 8)
    hm = hm.transpose(0, 1, 3, 2, 4).reshape(B, 1, H * 8, W * 8)
    return hm


def _interp_coords(xf, Wf, Wimg):
    gx = 2.0 * xf / (Wimg - 1) - 1.0
    ix = ((gx + 1.0) * Wf - 1.0) / 2.0
    x0 = jnp.floor(ix)
    return x0.astype(jnp.int32), ix - x0


def _cubic_weights(t):
    A = -0.75
    def k_out(s):
        return ((A * s - 5.0 * A) * s + 8.0 * A) * s - 4.0 * A
    def k_in(s):
        return ((A + 2.0) * s - (A + 3.0)) * s * s + 1.0
    return (k_out(t + 1.0), k_in(t), k_in(1.0 - t), k_out(2.0 - t))


def _axis_weights(idx, tfrac, taps_off, taps_w, Wf):
    col = jax.lax.broadcasted_iota(jnp.int32, (idx.shape[0], Wf), 1)
    acc = jnp.zeros((idx.shape[0], Wf), jnp.float32)
    for off, w in zip(taps_off, taps_w):
        xi = idx + off
        valid = (xi >= 0) & (xi < Wf)
        xc = jnp.clip(xi, 0, Wf - 1)
        acc = acc + jnp.where((col == xc) & valid, w, 0.0)
    return acc


def _row_gather(y_idx, mat):
    K = y_idx.shape[0]
    row = jax.lax.broadcasted_iota(jnp.int32, (K, mat.shape[0]), 1)
    oh = jnp.where(row == y_idx, 1.0, 0.0)
    return jax.lax.dot(oh, mat, preferred_element_type=jnp.float32, precision=jax.lax.Precision.HIGHEST)


def _col_select(rowvals, x_idx):
    col = jax.lax.broadcasted_iota(jnp.int32, rowvals.shape, 1)
    return jnp.sum(jnp.where(col == x_idx, rowvals, 0.0), axis=1, keepdims=True)


def _scores_body(xs_ref, ys_ref, vals_ref, hm_ref, out_ref):
    xs = xs_ref[0]
    ys = ys_ref[0]
    vals = vals_ref[0]

    x0, tx = _interp_coords(xs.astype(jnp.float32), 64, 512)
    y0, ty = _interp_coords(ys.astype(jnp.float32), 64, 512)
    hm = hm_ref[0]

    def corner_vals(rowvals, yi, xi):
        valid = (xi >= 0) & (xi < 64) & (yi >= 0) & (yi < 64)
        v = _col_select(rowvals, jnp.clip(xi, 0, 63))
        return jnp.where(valid, v, 0.0)

    rows0 = _row_gather(jnp.clip(y0, 0, 63), hm)
    rows1 = _row_gather(jnp.clip(y0 + 1, 0, 63), hm)
    v00 = corner_vals(rows0, y0, x0)
    v01 = corner_vals(rows0, y0, x0 + 1)
    v10 = corner_vals(rows1, y0 + 1, x0)
    v11 = corner_vals(rows1, y0 + 1, x0 + 1)
    s_bil = (v00 * ((1 - tx) * (1 - ty)) + v01 * (tx * (1 - ty))
             + v10 * ((1 - tx) * ty) + v11 * (tx * ty))

    near_ok = (xs < 511) & (ys < 511)
    s_near = jnp.where(near_ok, vals, 0.0)
    scores = s_near * s_bil
    scores = jnp.where((xs == 0) & (ys == 0), -1.0, scores)
    out_ref[0] = scores


def _pallas_scores(xs, ys, vals, hm):
    B, K = xs.shape
    grid = (B,)
    specs3 = pl.BlockSpec((1, K, 1), lambda i: (i, 0, 0))
    hspec = pl.BlockSpec((1, 64, 64), lambda i: (i, 0, 0))
    out = pl.pallas_call(
        _scores_body,
        grid=grid,
        in_specs=[specs3, specs3, specs3, hspec],
        out_specs=specs3,
        out_shape=jax.ShapeDtypeStruct((B, K, 1), jnp.float32),
    )(xs[..., None], ys[..., None], vals[..., None], hm)
    return out[..., 0]


def _bicubic_body(xs_ref, ys_ref, feats_ref, out_ref):
    xs = xs_ref[0]
    ys = ys_ref[0]
    K = xs.shape[0]

    x0, tx = _interp_coords(xs.astype(jnp.float32), 64, 512)
    y0, ty = _interp_coords(ys.astype(jnp.float32), 64, 512)
    cwx = _cubic_weights(tx)
    cwy = _cubic_weights(ty)
    wx = _axis_weights(x0, tx, (-1, 0, 1, 2), cwx, 64)
    wy = _axis_weights(y0, ty, (-1, 0, 1, 2), cwy, 64)

    CH = 64
    acc = jnp.zeros((K, CH), jnp.float32)
    CHUNK = 1024
    for s in range(0, 4096, CHUNK):
        rc = jax.lax.broadcasted_iota(jnp.int32, (64, CHUNK), 1) + s
        r_idx = rc // 64
        c_idx = rc % 64
        row64 = jax.lax.broadcasted_iota(jnp.int32, (64, CHUNK), 0)
        E = jnp.where(r_idx == row64, 1.0, 0.0)
        T = jnp.where(c_idx == row64, 1.0, 0.0)
        wyx = jax.lax.dot(wy, E, preferred_element_type=jnp.float32, precision=jax.lax.Precision.HIGHEST)
        wxx = jax.lax.dot(wx, T, preferred_element_type=jnp.float32, precision=jax.lax.Precision.HIGHEST)
        Wc = wyx * wxx
        acc = acc + jax.lax.dot_general(
            Wc, feats_ref[0, :, pl.ds(s, CHUNK)],
            (((1,), (1,)), ((), ())),
            preferred_element_type=jnp.float32,
            precision=jax.lax.Precision.HIGHEST)
    nrm = jnp.sqrt(jnp.sum(acc * acc, axis=1, keepdims=True))
    out_ref[0] = acc / jnp.maximum(nrm, 1e-12)


def _pallas_bicubic_norm(xs, ys, feats_c_rc):
    B, K = xs.shape
    grid = (B,)
    spec3 = pl.BlockSpec((1, K, 1), lambda i: (i, 0, 0))
    fspec = pl.BlockSpec((1, 64, 4096), lambda i: (i, 0, 0))
    ospec = pl.BlockSpec((1, K, 64), lambda i: (i, 0, 0))
    return pl.pallas_call(
        _bicubic_body,
        grid=grid,
        in_specs=[spec3, spec3, fspec],
        out_specs=ospec,
        out_shape=jax.ShapeDtypeStruct((B, K, 64), jnp.float32),
    )(xs[..., None], ys[..., None], feats_c_rc)


def kernel(images, params):
    b, v, c, h, w = images.shape
    x = images.reshape(b * v, c, h, w)
    feats, kpts_logits, heatmap = _backbone(x, params)

    sf = feats.reshape(b, v * feats.shape[1], feats.shape[2], feats.shape[3])
    t = jax.nn.relu(_conv(sf, params['sh_w1'], params['sh_b1']))
    t = _conv(t, params['sh_wdw'], params['sh_bdw'], stride=4, pad=3, groups=64)
    t = jax.nn.relu(_conv(t, params['sh_w2'], params['sh_b2']))
    t = t.mean(axis=(2, 3))
    t_scale = t @ params['sh_wl'].T + params['sh_bl']

    kh = _kpts_heatmap(kpts_logits)
    Bv = kh.shape[0]
    flat = kh.reshape(Bv, -1)
    vals, inds = jax.lax.top_k(flat, TOPK_N)
    ys = inds // w
    xs = inds % w

    kpts_i = jnp.stack([xs, ys], axis=-1)
    kf = kpts_i.astype(jnp.float32)
    s_near = _grid_sample_pts_xla(kh, kf, h, w, 'nearest')[..., 0]
    s_bil = _grid_sample_pts_xla(heatmap, kf, h, w, 'bilinear')[..., 0]
    scores = s_near * s_bil
    scores = jnp.where(jnp.all(kpts_i == 0, axis=-1), -1.0, scores)
    idxs = jnp.argsort(-scores, axis=-1)
    kx = jnp.take_along_axis(xs, idxs, axis=-1)
    ky = jnp.take_along_axis(ys, idxs, axis=-1)
    scores = jnp.take_along_axis(scores, idxs, axis=-1)

    feats_c_rc = feats.reshape(Bv, 64, 4096)
    fs = _pallas_bicubic_norm(kx, ky, feats_c_rc)

    kpts = jnp.stack([kx, ky], axis=-1).astype(jnp.float32)
    return fs, scores, kpts, t_scale
